# baseline (speedup 1.0000x reference)
import jax
import jax.numpy as jnp
from jax import lax
from jax.experimental import pallas as pl
from jax.experimental.pallas import tpu as pltpu
from jax.experimental.pallas import tpu_sc as plsc

B = 16384
M = 20
D = 32
NC = 2
NS = 16
NW = NC * NS
QPW = B // NW
C = 64
NCHUNK = QPW // C
IPC = C * M
GW = 128
NSUB = IPC // GW


def _ges_body(qid_hbm, qcat_hbm, qbrand_hbm, match_hbm,
              id_t, cat_t, brand_t, out_t, out_hbm,
              qi_idx, qc_idx, qb_idx, mi_idx,
              id_rows, cat_rows, br_rows, m_rows, logits, isem, sem):
    wid = lax.axis_index("s") * NC + lax.axis_index("c")

    def chunk_body(c, carry):
        b0 = wid * QPW + c * C
        idescs = [
            pltpu.async_copy(qid_hbm.at[pl.ds(b0, C)], qi_idx, isem),
            pltpu.async_copy(qcat_hbm.at[pl.ds(b0, C)], qc_idx, isem),
            pltpu.async_copy(qbrand_hbm.at[pl.ds(b0, C)], qb_idx, isem),
        ]
        for j in range(NSUB):
            idescs.append(pltpu.async_copy(
                match_hbm.at[pl.ds(b0 * M + j * GW, GW)], mi_idx.at[j], isem))
        for d_ in idescs:
            d_.wait()
        descs = [
            pltpu.async_copy(id_t.at[qi_idx], id_rows, sem),
            pltpu.async_copy(cat_t.at[qc_idx], cat_rows, sem),
            pltpu.async_copy(brand_t.at[qb_idx], br_rows, sem),
        ]
        for j in range(NSUB):
            descs.append(pltpu.async_copy(
                out_t.at[mi_idx.at[j]], m_rows.at[pl.ds(j * GW, GW)], sem))
        for d_ in descs:
            d_.wait()

        third = jnp.float32(1.0 / 3.0)
        lane = lax.iota(jnp.int32, 16)

        def g_body(g, carry_q):
            accs = [jnp.zeros((16,), jnp.float32) for _ in range(5)]
            for bi in range(4):
                b = g * 4 + bi
                h0 = (id_rows[b, pl.ds(0, 16)] + cat_rows[b, pl.ds(0, 16)]
                      + br_rows[b, pl.ds(0, 16)]) * third
                h1 = (id_rows[b, pl.ds(16, 16)] + cat_rows[b, pl.ds(16, 16)]
                      + br_rows[b, pl.ds(16, 16)]) * third
                for m in range(M):
                    row = b * M + m
                    p = (m_rows[row, pl.ds(0, 16)] * h0
                         + m_rows[row, pl.ds(16, 16)] * h1)
                    s = jnp.sum(p)
                    k, ln = divmod(bi * M + m, 16)
                    accs[k] = jnp.where(lane == ln, s, accs[k])
            for k in range(5):
                logits[pl.ds(g * 80 + k * 16, 16)] = accs[k]
            return carry_q

        lax.fori_loop(0, C // 4, g_body, 0)
        pltpu.sync_copy(logits, out_hbm.at[pl.ds(b0 * M, IPC)])
        return carry

    lax.fori_loop(0, NCHUNK, chunk_body, 0)


_TBLK = 8192


def _tc_transpose_body(in_ref, out_ref):
    eye = jnp.eye(D, dtype=jnp.float32)
    out_ref[...] = jax.lax.dot_general(
        in_ref[...], eye, (((0,), (0,)), ((), ())),
        preferred_element_type=jnp.float32)


def _to_row_major(table):
    v = table.shape[0]
    t = jnp.swapaxes(table, 0, 1)
    grid = (v + _TBLK - 1) // _TBLK
    return pl.pallas_call(
        _tc_transpose_body,
        grid=(grid,),
        in_specs=[pl.BlockSpec((D, _TBLK), lambda i: (0, i))],
        out_specs=pl.BlockSpec((_TBLK, D), lambda i: (i, 0)),
        out_shape=jax.ShapeDtypeStruct((v, D), jnp.float32),
    )(t)


def kernel(query_item_id, query_cat_id, query_brand_id, match,
           id_table, cat_table, brand_table, out_table):
    id_table = _to_row_major(id_table)
    cat_table = _to_row_major(cat_table)
    brand_table = _to_row_major(brand_table)
    out_table = _to_row_major(out_table)
    qid = query_item_id.reshape(B).astype(jnp.int32)
    qcat = query_cat_id.reshape(B).astype(jnp.int32)
    qbrand = query_brand_id.reshape(B).astype(jnp.int32)
    match_r = match.reshape(B * M).astype(jnp.int32)

    mesh = plsc.VectorSubcoreMesh(
        core_axis_name="c", subcore_axis_name="s",
        num_cores=NC, num_subcores=NS)
    run = pl.kernel(
        _ges_body,
        out_type=jax.ShapeDtypeStruct((B * M,), jnp.float32),
        mesh=mesh,
        compiler_params=pltpu.CompilerParams(
            needs_layout_passes=False, use_tc_tiling_on_sc=False),
        scratch_types=[
            pltpu.VMEM((C,), jnp.int32),
            pltpu.VMEM((C,), jnp.int32),
            pltpu.VMEM((C,), jnp.int32),
            pltpu.VMEM((NSUB, GW), jnp.int32),
            pltpu.VMEM((C, D), jnp.float32),
            pltpu.VMEM((C, D), jnp.float32),
            pltpu.VMEM((C, D), jnp.float32),
            pltpu.VMEM((IPC, D), jnp.float32),
            pltpu.VMEM((IPC,), jnp.float32),
            pltpu.SemaphoreType.DMA,
            pltpu.SemaphoreType.DMA,
        ],
    )
    flat = run(qid, qcat, qbrand, match_r,
               id_table, cat_table, brand_table, out_table)
    return flat.reshape(B, M)


# speedup vs baseline: 2.0420x; 2.0420x over previous
import jax
import jax.numpy as jnp
from jax import lax
from jax.experimental import pallas as pl
from jax.experimental.pallas import tpu as pltpu
from jax.experimental.pallas import tpu_sc as plsc

B = 16384
M = 20
D = 32
NC = 2
NS = 16
NW = NC * NS
QPW = B // NW
C = 64
NCHUNK = QPW // C
IPC = C * M
GW = 128
NSUB = IPC // GW


def _ges_body(qri_hbm, qrc_hbm, qrb_hbm, match_hbm, out_t, out_hbm,
              mi_idx, id_rows, cat_rows, br_rows, m_rows, logits, isem, sem):
    wid = lax.axis_index("s") * NC + lax.axis_index("c")

    def chunk_body(c, carry):
        b0 = wid * QPW + c * C
        idescs = [
            pltpu.async_copy(qri_hbm.at[pl.ds(b0, C)], id_rows, isem),
            pltpu.async_copy(qrc_hbm.at[pl.ds(b0, C)], cat_rows, isem),
            pltpu.async_copy(qrb_hbm.at[pl.ds(b0, C)], br_rows, isem),
        ]
        for j in range(NSUB):
            idescs.append(pltpu.async_copy(
                match_hbm.at[pl.ds(b0 * M + j * GW, GW)], mi_idx.at[j], isem))
        for d_ in idescs:
            d_.wait()
        descs = []
        for j in range(NSUB):
            descs.append(pltpu.async_copy(
                out_t.at[mi_idx.at[j]], m_rows.at[pl.ds(j * GW, GW)], sem))
        for d_ in descs:
            d_.wait()

        third = jnp.float32(1.0 / 3.0)
        lane = lax.iota(jnp.int32, 16)

        def g_body(g, carry_q):
            accs = [jnp.zeros((16,), jnp.float32) for _ in range(5)]
            for bi in range(4):
                b = g * 4 + bi
                h0 = (id_rows[b, pl.ds(0, 16)] + cat_rows[b, pl.ds(0, 16)]
                      + br_rows[b, pl.ds(0, 16)]) * third
                h1 = (id_rows[b, pl.ds(16, 16)] + cat_rows[b, pl.ds(16, 16)]
                      + br_rows[b, pl.ds(16, 16)]) * third
                for m in range(M):
                    row = b * M + m
                    p = (m_rows[row, pl.ds(0, 16)] * h0
                         + m_rows[row, pl.ds(16, 16)] * h1)
                    s = jnp.sum(p)
                    k, ln = divmod(bi * M + m, 16)
                    accs[k] = jnp.where(lane == ln, s, accs[k])
            for k in range(5):
                logits[pl.ds(g * 80 + k * 16, 16)] = accs[k]
            return carry_q

        lax.fori_loop(0, C // 4, g_body, 0)
        pltpu.sync_copy(logits, out_hbm.at[pl.ds(b0 * M, IPC)])
        return carry

    lax.fori_loop(0, NCHUNK, chunk_body, 0)


def kernel(query_item_id, query_cat_id, query_brand_id, match,
           id_table, cat_table, brand_table, out_table):
    qid = query_item_id.reshape(B).astype(jnp.int32)
    qcat = query_cat_id.reshape(B).astype(jnp.int32)
    qbrand = query_brand_id.reshape(B).astype(jnp.int32)
    match_r = match.reshape(B * M).astype(jnp.int32)
    qrows_i = jnp.take(id_table, qid, axis=0)
    qrows_c = jnp.take(cat_table, qcat, axis=0)
    qrows_b = jnp.take(brand_table, qbrand, axis=0)

    mesh = plsc.VectorSubcoreMesh(
        core_axis_name="c", subcore_axis_name="s",
        num_cores=NC, num_subcores=NS)
    run = pl.kernel(
        _ges_body,
        out_type=jax.ShapeDtypeStruct((B * M,), jnp.float32),
        mesh=mesh,
        compiler_params=pltpu.CompilerParams(
            needs_layout_passes=False, use_tc_tiling_on_sc=False),
        scratch_types=[
            pltpu.VMEM((NSUB, GW), jnp.int32),
            pltpu.VMEM((C, D), jnp.float32),
            pltpu.VMEM((C, D), jnp.float32),
            pltpu.VMEM((C, D), jnp.float32),
            pltpu.VMEM((IPC, D), jnp.float32),
            pltpu.VMEM((IPC,), jnp.float32),
            pltpu.SemaphoreType.DMA,
            pltpu.SemaphoreType.DMA,
        ],
    )
    flat = run(qrows_i, qrows_c, qrows_b, match_r, out_table)
    return flat.reshape(B, M)
